# baseline (speedup 1.0000x reference)
import functools

import jax
import jax.numpy as jnp
from jax import lax
from jax.experimental import pallas as pl
from jax.experimental.pallas import tpu as pltpu
from jax.experimental.pallas import tpu_sc as plsc

_B = 1024
_V = 100000


_VR = 5000
_NVJ = _V // _VR
_GPB = -(-_B // _NVJ)


def _tc_body(tgt_ref, x_ref, x_any, lse_ref, tv_ref, m_ref, s_ref, win, gsem):
    j = pl.program_id(0)

    @pl.when(j == 0)
    def _init():
        m_ref[...] = jnp.full((1, _B), -jnp.inf, jnp.float32)
        s_ref[...] = jnp.zeros((1, _B), jnp.float32)

    for q in range(_GPB):
        k = j * _GPB + q

        @pl.when(k < _B)
        def _():
            cb = pl.multiple_of((k // 128) * 128, 128)
            pltpu.make_async_copy(
                x_any.at[pl.ds(tgt_ref[k], 1), pl.ds(cb, 128)],
                win.at[pl.ds(k, 1), :],
                gsem,
            ).start()

    x = x_ref[...]
    bm = jnp.max(x, axis=0, keepdims=True)
    m_old = m_ref[...]
    m_new = jnp.maximum(m_old, bm)
    e_sum = jnp.sum(jnp.exp(x - m_new), axis=0, keepdims=True)
    s_new = s_ref[...] * jnp.exp(m_old - m_new) + e_sum
    s_ref[...] = s_new
    m_ref[...] = m_new

    @pl.when(j == _NVJ - 1)
    def _fin():
        lse_ref[...] = m_new + jnp.log(s_new)
        pltpu.make_async_copy(
            x_any.at[pl.ds(0, _B), pl.ds(0, 128)], win, gsem
        ).wait()
        rows = lax.broadcasted_iota(jnp.int32, (_B, 128), 0)
        cols = lax.broadcasted_iota(jnp.int32, (_B, 128), 1)
        sel = cols == lax.rem(rows, 128)
        tv_ref[...] = jnp.sum(
            jnp.where(sel, win[...], 0.0), axis=1, keepdims=True
        )


_tc_main = pl.pallas_call(
    _tc_body,
    grid_spec=pltpu.PrefetchScalarGridSpec(
        num_scalar_prefetch=1,
        grid=(_NVJ,),
        in_specs=[
            pl.BlockSpec((_VR, _B), lambda j, tgt_ref: (j, 0)),
            pl.BlockSpec(memory_space=pl.ANY),
        ],
        out_specs=[
            pl.BlockSpec((1, _B), lambda j, tgt_ref: (0, 0)),
            pl.BlockSpec((_B, 1), lambda j, tgt_ref: (0, 0)),
        ],
        scratch_shapes=[
            pltpu.VMEM((1, _B), jnp.float32),
            pltpu.VMEM((1, _B), jnp.float32),
            pltpu.VMEM((_B, 128), jnp.float32),
            pltpu.SemaphoreType.DMA,
        ],
    ),
    out_shape=[
        jax.ShapeDtypeStruct((1, _B), jnp.float32),
        jax.ShapeDtypeStruct((_B, 1), jnp.float32),
    ],
    compiler_params=pltpu.CompilerParams(
        dimension_semantics=("arbitrary",),
        vmem_limit_bytes=57 * 1024 * 1024,
    ),
)


_NC = 2
_NS = 16


def _sc_mine_body(lse, tv, out, lse_v, tv_v, out_v):
    c = lax.axis_index("c")
    s = lax.axis_index("s")
    wid = s * _NC + c

    @pl.when(wid == 0)
    def _():
        pltpu.sync_copy(lse, lse_v)
        pltpu.sync_copy(tv, tv_v)
        acc = jnp.zeros((16,), jnp.float32)
        for u in range(_B // 16):
            acc = acc + (lse_v[pl.ds(u * 16, 16)] - tv_v[pl.ds(u * 16, 16)])
        total = lax.reduce_sum_p.bind(acc, axes=(0,))
        out_v[...] = jnp.full((16,), total * (1.0 / _B), jnp.float32)
        pltpu.sync_copy(out_v, out)


@functools.cache
def _sc_mine():
    return pl.kernel(
        _sc_mine_body,
        out_type=jax.ShapeDtypeStruct((16,), jnp.float32),
        mesh=plsc.VectorSubcoreMesh(
            core_axis_name="c", subcore_axis_name="s",
            num_cores=_NC, num_subcores=_NS,
        ),
        scratch_types=[
            pltpu.VMEM((_B,), jnp.float32),
            pltpu.VMEM((_B,), jnp.float32),
            pltpu.VMEM((16,), jnp.float32),
        ],
        compiler_params=pltpu.CompilerParams(needs_layout_passes=False),
    )


def kernel(inputs, targets):
    xt = inputs.T
    tgt = targets.astype(jnp.int32)
    lse, tv = _tc_main(tgt, xt, xt)
    loss = _sc_mine()(lse.reshape(_B), tv.reshape(_B))
    return loss[0]


# speedup vs baseline: 1.0109x; 1.0109x over previous
import functools

import jax
import jax.numpy as jnp
from jax import lax
from jax.experimental import pallas as pl
from jax.experimental.pallas import tpu as pltpu
from jax.experimental.pallas import tpu_sc as plsc

_B = 1024
_V = 100000


_VR = 4000
_NVJ = _V // _VR
_GPB = -(-_B // _NVJ)


def _tc_body(tgt_ref, x_ref, x_any, lse_ref, tv_ref, m_ref, s_ref, win, gsem):
    j = pl.program_id(0)

    @pl.when(j == 0)
    def _init():
        m_ref[...] = jnp.full((1, _B), -jnp.inf, jnp.float32)
        s_ref[...] = jnp.zeros((1, _B), jnp.float32)

    for q in range(_GPB):
        k = j * _GPB + q

        @pl.when(k < _B)
        def _():
            cb = pl.multiple_of((k // 128) * 128, 128)
            pltpu.make_async_copy(
                x_any.at[pl.ds(tgt_ref[k], 1), pl.ds(cb, 128)],
                win.at[pl.ds(k, 1), :],
                gsem,
            ).start()

    x = x_ref[...]
    bm = jnp.max(x, axis=0, keepdims=True)
    m_old = m_ref[...]
    m_new = jnp.maximum(m_old, bm)
    e_sum = jnp.sum(jnp.exp(x - m_new), axis=0, keepdims=True)
    s_new = s_ref[...] * jnp.exp(m_old - m_new) + e_sum
    s_ref[...] = s_new
    m_ref[...] = m_new

    @pl.when(j == _NVJ - 1)
    def _fin():
        lse_ref[...] = m_new + jnp.log(s_new)
        pltpu.make_async_copy(
            x_any.at[pl.ds(0, _B), pl.ds(0, 128)], win, gsem
        ).wait()
        rows = lax.broadcasted_iota(jnp.int32, (_B, 128), 0)
        cols = lax.broadcasted_iota(jnp.int32, (_B, 128), 1)
        sel = cols == lax.rem(rows, 128)
        tv_ref[...] = jnp.sum(
            jnp.where(sel, win[...], 0.0), axis=1, keepdims=True
        )


_tc_main = pl.pallas_call(
    _tc_body,
    grid_spec=pltpu.PrefetchScalarGridSpec(
        num_scalar_prefetch=1,
        grid=(_NVJ,),
        in_specs=[
            pl.BlockSpec((_VR, _B), lambda j, tgt_ref: (j, 0)),
            pl.BlockSpec(memory_space=pl.ANY),
        ],
        out_specs=[
            pl.BlockSpec((1, _B), lambda j, tgt_ref: (0, 0)),
            pl.BlockSpec((_B, 1), lambda j, tgt_ref: (0, 0)),
        ],
        scratch_shapes=[
            pltpu.VMEM((1, _B), jnp.float32),
            pltpu.VMEM((1, _B), jnp.float32),
            pltpu.VMEM((_B, 128), jnp.float32),
            pltpu.SemaphoreType.DMA,
        ],
    ),
    out_shape=[
        jax.ShapeDtypeStruct((1, _B), jnp.float32),
        jax.ShapeDtypeStruct((_B, 1), jnp.float32),
    ],
    compiler_params=pltpu.CompilerParams(
        dimension_semantics=("arbitrary",),
        vmem_limit_bytes=57 * 1024 * 1024,
    ),
)


_NC = 2
_NS = 16


def _sc_mine_body(lse, tv, out, lse_v, tv_v, out_v):
    c = lax.axis_index("c")
    s = lax.axis_index("s")
    wid = s * _NC + c

    @pl.when(wid == 0)
    def _():
        pltpu.sync_copy(lse, lse_v)
        pltpu.sync_copy(tv, tv_v)
        acc = jnp.zeros((16,), jnp.float32)
        for u in range(_B // 16):
            acc = acc + (lse_v[pl.ds(u * 16, 16)] - tv_v[pl.ds(u * 16, 16)])
        total = lax.reduce_sum_p.bind(acc, axes=(0,))
        out_v[...] = jnp.full((16,), total * (1.0 / _B), jnp.float32)
        pltpu.sync_copy(out_v, out)


@functools.cache
def _sc_mine():
    return pl.kernel(
        _sc_mine_body,
        out_type=jax.ShapeDtypeStruct((16,), jnp.float32),
        mesh=plsc.VectorSubcoreMesh(
            core_axis_name="c", subcore_axis_name="s",
            num_cores=_NC, num_subcores=_NS,
        ),
        scratch_types=[
            pltpu.VMEM((_B,), jnp.float32),
            pltpu.VMEM((_B,), jnp.float32),
            pltpu.VMEM((16,), jnp.float32),
        ],
        compiler_params=pltpu.CompilerParams(needs_layout_passes=False),
    )


def kernel(inputs, targets):
    xt = inputs.T
    tgt = targets.astype(jnp.int32)
    lse, tv = _tc_main(tgt, xt, xt)
    loss = _sc_mine()(lse.reshape(_B), tv.reshape(_B))
    return loss[0]
